# baseline (speedup 1.0000x reference)
import functools

import jax
import jax.numpy as jnp
from jax import lax
from jax.experimental import pallas as pl
from jax.experimental.pallas import tpu as pltpu
from jax.experimental.pallas import tpu_sc as plsc

_THRESH = 1e-10

_OFF_ALUT = 1 * 128
_OFF_RLUT = 6 * 128
_OFF_AB = 11 * 128
_OFF_RB = 16 * 128
_COL_ZB = 8

_NC, _NS, _L = 2, 16, 16
_NW = _NC * _NS
_ROWS, _COLS = 4096, 2048
_ROWS_W = _ROWS // _NW
_CROWS = 4
_NCHUNK = _ROWS_W // _CROWS
_TRIPS = _CROWS * _COLS
_GROUPS = _TRIPS // _L


def _lut_body(zb_ref, ab_ref, rb_ref, babs_ref, out_ref):
    zb = zb_ref[...]
    ab = ab_ref[...]
    rb = rb_ref[...]
    babs = babs_ref[...]

    ri = lax.broadcasted_iota(jnp.int32, (5, 6), 0)
    ci = lax.broadcasted_iota(jnp.int32, (5, 6), 1)
    min_r = 0.5 * jnp.sum(jnp.where((ri == 1) & (ci < 2), babs, 0.0))
    max_r = 0.5 * jnp.sum(jnp.where((ri == 4) & (ci >= 4), babs, 0.0))

    one1 = jnp.ones((1, 1), jnp.float32)
    zero1 = jnp.zeros((1, 1), jnp.float32)
    z5 = jnp.concatenate([zero1, 0.5 * (zb[:, 1:] + zb[:, :-1]), one1], axis=1)
    row0 = jnp.concatenate(
        [z5, jnp.zeros((1, 3), jnp.float32), zb, jnp.zeros((1, 116), jnp.float32)], axis=1)

    ones5 = jnp.ones((5, 1), jnp.float32)
    zeros5 = jnp.zeros((5, 1), jnp.float32)
    a_lut = jnp.concatenate(
        [zeros5, 0.5 * (ab[:, 1:] + ab[:, :-1]), ones5,
         jnp.zeros((5, 111), jnp.float32)], axis=1)
    r_mid = jnp.concatenate(
        [zeros5, 0.5 * (rb[:, 1:] + rb[:, :-1]), ones5], axis=1)
    r_lut = jnp.clip(jnp.log(r_mid * max_r + min_r + _THRESH), 0.0, 1.0)
    r_lut = jnp.concatenate([r_lut, jnp.zeros((5, 111), jnp.float32)], axis=1)
    ab_pad = jnp.concatenate([ab, jnp.zeros((5, 112), jnp.float32)], axis=1)
    rb_pad = jnp.concatenate([rb, jnp.zeros((5, 112), jnp.float32)], axis=1)

    out_ref[...] = jnp.concatenate(
        [row0, a_lut, r_lut, ab_pad, rb_pad, jnp.zeros((11, 128), jnp.float32)],
        axis=0)


def _build_pack(zb2, ab, rb, babs):
    return pl.pallas_call(
        _lut_body,
        out_shape=jax.ShapeDtypeStruct((32, 128), jnp.float32),
    )(zb2, ab, rb, babs)


def _search5(lut_v, off, v):
    pos = jnp.zeros((_L,), jnp.int32)
    for s in (8, 4, 2, 1):
        probe = plsc.load_gather(lut_v, [off + (pos + (s - 1))])
        pos = jnp.where(probe < v, pos + s, pos)
    probe = plsc.load_gather(lut_v, [off + pos])
    return jnp.where(probe < v, pos + 1, pos)


def _sc_body(x_hbm, pack_hbm, o_hbm, lut_v,
             iz0, ia0, ir0, iz1, ia1, ir1,
             oz0, oa0, or0, oz1, oa1, or1,
             sem_in0, sem_in1, sem_out0, sem_out1):
    cid = lax.axis_index("c")
    sid = lax.axis_index("s")
    wid = sid * _NC + cid
    base_row = wid * _ROWS_W

    pltpu.sync_copy(pack_hbm, lut_v)

    zeros16 = jnp.zeros((_L,), jnp.int32)
    zbs = [plsc.load_gather(lut_v, [zeros16 + (_COL_ZB + j)])
           for j in range(4)]

    ins = ((iz0, ia0, ir0), (iz1, ia1, ir1))
    outs = ((oz0, oa0, or0), (oz1, oa1, or1))
    sem_ins = (sem_in0, sem_in1)
    sem_outs = (sem_out0, sem_out1)

    def in_slice(c, ch):
        return x_hbm.at[ch, pl.ds(base_row + c * _CROWS, _CROWS)]

    def out_slice(c, ch):
        return o_hbm.at[ch, pl.ds(base_row + c * _CROWS, _CROWS)]

    def start_in(c, b):
        for ch in range(3):
            pltpu.async_copy(in_slice(c, ch), ins[b][ch], sem_ins[b])

    def wait_in(c, b):
        for ch in range(3):
            pltpu.make_async_copy(in_slice(c, ch), ins[b][ch], sem_ins[b]).wait()

    def start_out(c, b):
        for ch in range(3):
            pltpu.async_copy(outs[b][ch], out_slice(c, ch), sem_outs[b])

    def wait_out(c, b):
        for ch in range(3):
            pltpu.make_async_copy(outs[b][ch], out_slice(c, ch), sem_outs[b]).wait()

    start_in(0, 0)
    start_in(1, 1)

    def compute_chunk(bufs_in, bufs_out):
        bz, ba, br = bufs_in
        oz, oa, orr = bufs_out

        @plsc.parallel_loop(0, _GROUPS, step=1, unroll=4)
        def _grp(g):
            rr = lax.shift_right_logical(g, 7)
            cc = jnp.bitwise_and(g, 127) * _L
            z = bz[rr, pl.ds(cc, _L)]
            a = ba[rr, pl.ds(cc, _L)]
            r = br[rr, pl.ds(cc, _L)]

            zbin = (z > zbs[0]).astype(jnp.int32)
            for j in range(1, 4):
                zbin = zbin + (z > zbs[j]).astype(jnp.int32)
            zoff = zbin * 128

            new_z = plsc.load_gather(lut_v, [zbin])
            posa = _search5(lut_v, zoff + _OFF_AB, a)
            new_a = plsc.load_gather(lut_v, [(zoff + _OFF_ALUT) + posa])
            posr = _search5(lut_v, zoff + _OFF_RB, r)
            new_r = plsc.load_gather(lut_v, [(zoff + _OFF_RLUT) + posr])

            oz[rr, pl.ds(cc, _L)] = new_z
            oa[rr, pl.ds(cc, _L)] = new_a
            orr[rr, pl.ds(cc, _L)] = new_r

    def outer(i, _):
        for b in range(2):
            c = 2 * i + b
            wait_in(c, b)

            @pl.when(c >= 2)
            def _():
                wait_out(c - 2, b)

            compute_chunk(ins[b], outs[b])
            start_out(c, b)

            @pl.when(c + 2 < _NCHUNK)
            def _():
                start_in(c + 2, b)
        return 0

    lax.fori_loop(0, _NCHUNK // 2, outer, 0)

    wait_out(_NCHUNK - 2, 0)
    wait_out(_NCHUNK - 1, 1)


_CBUF = pltpu.VMEM((_CROWS, _COLS), jnp.float32)

_sc_run = functools.partial(
    pl.kernel,
    out_type=jax.ShapeDtypeStruct((3, _ROWS, _COLS), jnp.float32),
    mesh=plsc.VectorSubcoreMesh(core_axis_name="c", subcore_axis_name="s",
                                num_cores=_NC, num_subcores=_NS),
    compiler_params=pltpu.CompilerParams(needs_layout_passes=False,
                                         use_tc_tiling_on_sc=False),
    scratch_types=[
        pltpu.VMEM((32 * 128,), jnp.float32),
        _CBUF, _CBUF, _CBUF, _CBUF, _CBUF, _CBUF,
        _CBUF, _CBUF, _CBUF, _CBUF, _CBUF, _CBUF,
        pltpu.SemaphoreType.DMA,
        pltpu.SemaphoreType.DMA,
        pltpu.SemaphoreType.DMA,
        pltpu.SemaphoreType.DMA,
    ],
)(_sc_body)


def kernel(x, z_boundaries, alpha_boundaries, r_boundaries_log, boundaries_abs_r):
    pack = _build_pack(z_boundaries.reshape(1, 4), alpha_boundaries,
                       r_boundaries_log, boundaries_abs_r)
    out = _sc_run(x.transpose(2, 0, 1), pack.reshape(-1))
    return out.transpose(1, 2, 0)


# speedup vs baseline: 1.1091x; 1.1091x over previous
import functools

import jax
import jax.numpy as jnp
from jax import lax
from jax.experimental import pallas as pl
from jax.experimental.pallas import tpu as pltpu
from jax.experimental.pallas import tpu_sc as plsc

_THRESH = 1e-10

_OFF_ALUT = 1 * 128
_OFF_RLUT = 6 * 128
_OFF_AB = 11 * 128
_OFF_RB = 16 * 128
_COL_ZB = 8

_NC, _NS, _L = 2, 16, 16
_NW = _NC * _NS
_ROWS, _COLS = 4096, 2048
_ROWS_W = _ROWS // _NW
_CROWS = 4
_NCHUNK = _ROWS_W // _CROWS
_TRIPS = _CROWS * _COLS
_GROUPS = _TRIPS // _L


def _lut_body(zb_ref, ab_ref, rb_ref, babs_ref, out_ref):
    zb = zb_ref[...]
    ab = ab_ref[...]
    rb = rb_ref[...]
    babs = babs_ref[...]

    ri = lax.broadcasted_iota(jnp.int32, (5, 6), 0)
    ci = lax.broadcasted_iota(jnp.int32, (5, 6), 1)
    min_r = 0.5 * jnp.sum(jnp.where((ri == 1) & (ci < 2), babs, 0.0))
    max_r = 0.5 * jnp.sum(jnp.where((ri == 4) & (ci >= 4), babs, 0.0))

    one1 = jnp.ones((1, 1), jnp.float32)
    zero1 = jnp.zeros((1, 1), jnp.float32)
    z5 = jnp.concatenate([zero1, 0.5 * (zb[:, 1:] + zb[:, :-1]), one1], axis=1)
    row0 = jnp.concatenate(
        [z5, jnp.zeros((1, 3), jnp.float32), zb, jnp.zeros((1, 116), jnp.float32)], axis=1)

    ones5 = jnp.ones((5, 1), jnp.float32)
    zeros5 = jnp.zeros((5, 1), jnp.float32)
    a_lut = jnp.concatenate(
        [zeros5, 0.5 * (ab[:, 1:] + ab[:, :-1]), ones5,
         jnp.zeros((5, 111), jnp.float32)], axis=1)
    r_mid = jnp.concatenate(
        [zeros5, 0.5 * (rb[:, 1:] + rb[:, :-1]), ones5], axis=1)
    r_lut = jnp.clip(jnp.log(r_mid * max_r + min_r + _THRESH), 0.0, 1.0)
    r_lut = jnp.concatenate([r_lut, jnp.zeros((5, 111), jnp.float32)], axis=1)
    ab_pad = jnp.concatenate([ab, jnp.zeros((5, 112), jnp.float32)], axis=1)
    rb_pad = jnp.concatenate([rb, jnp.zeros((5, 112), jnp.float32)], axis=1)

    out_ref[...] = jnp.concatenate(
        [row0, a_lut, r_lut, ab_pad, rb_pad, jnp.zeros((11, 128), jnp.float32)],
        axis=0)


def _build_pack(zb2, ab, rb, babs):
    return pl.pallas_call(
        _lut_body,
        out_shape=jax.ShapeDtypeStruct((32, 128), jnp.float32),
    )(zb2, ab, rb, babs)


def _search5(lut_v, off, v):
    pos = jnp.zeros((_L,), jnp.int32)
    for s in (8, 4, 2, 1):
        probe = plsc.load_gather(lut_v, [off + (pos + (s - 1))])
        pos = jnp.where(probe < v, pos + s, pos)
    probe = plsc.load_gather(lut_v, [off + pos])
    return jnp.where(probe < v, pos + 1, pos)


def _sc_body(x_hbm, pack_hbm, o_hbm, lut_v,
             iz0, ia0, ir0, iz1, ia1, ir1,
             oz0, oa0, or0, oz1, oa1, or1,
             sem_in0, sem_in1, sem_out0, sem_out1):
    cid = lax.axis_index("c")
    sid = lax.axis_index("s")
    wid = sid * _NC + cid
    base_row = wid * _ROWS_W

    pltpu.sync_copy(pack_hbm, lut_v)

    zeros16 = jnp.zeros((_L,), jnp.int32)
    zbs = [plsc.load_gather(lut_v, [zeros16 + (_COL_ZB + j)])
           for j in range(4)]

    ins = ((iz0, ia0, ir0), (iz1, ia1, ir1))
    outs = ((oz0, oa0, or0), (oz1, oa1, or1))
    sem_ins = (sem_in0, sem_in1)
    sem_outs = (sem_out0, sem_out1)

    def in_slice(c, ch):
        return x_hbm.at[ch, pl.ds(base_row + c * _CROWS, _CROWS)]

    def out_slice(c, ch):
        return o_hbm.at[ch, pl.ds(base_row + c * _CROWS, _CROWS)]

    def start_in(c, b):
        for ch in range(3):
            pltpu.async_copy(in_slice(c, ch), ins[b][ch], sem_ins[b])

    def wait_in(c, b):
        for ch in range(3):
            pltpu.make_async_copy(in_slice(c, ch), ins[b][ch], sem_ins[b]).wait()

    def start_out(c, b):
        for ch in range(3):
            pltpu.async_copy(outs[b][ch], out_slice(c, ch), sem_outs[b])

    def wait_out(c, b):
        for ch in range(3):
            pltpu.make_async_copy(outs[b][ch], out_slice(c, ch), sem_outs[b]).wait()

    start_in(0, 0)
    start_in(1, 1)

    def compute_chunk(bufs_in, bufs_out):
        bz, ba, br = bufs_in
        oz, oa, orr = bufs_out

        @plsc.parallel_loop(0, _GROUPS, step=1, unroll=3)
        def _grp(g):
            rr = lax.shift_right_logical(g, 7)
            cc = jnp.bitwise_and(g, 127) * _L
            z = bz[rr, pl.ds(cc, _L)]
            a = ba[rr, pl.ds(cc, _L)]
            r = br[rr, pl.ds(cc, _L)]

            zbin = (z > zbs[0]).astype(jnp.int32)
            for j in range(1, 4):
                zbin = zbin + (z > zbs[j]).astype(jnp.int32)
            zoff = zbin * 128

            new_z = plsc.load_gather(lut_v, [zbin])
            posa = _search5(lut_v, zoff + _OFF_AB, a)
            new_a = plsc.load_gather(lut_v, [(zoff + _OFF_ALUT) + posa])
            posr = _search5(lut_v, zoff + _OFF_RB, r)
            new_r = plsc.load_gather(lut_v, [(zoff + _OFF_RLUT) + posr])

            oz[rr, pl.ds(cc, _L)] = new_z
            oa[rr, pl.ds(cc, _L)] = new_a
            orr[rr, pl.ds(cc, _L)] = new_r

    def outer(i, _):
        for b in range(2):
            c = 2 * i + b
            wait_in(c, b)

            @pl.when(c >= 2)
            def _():
                wait_out(c - 2, b)

            compute_chunk(ins[b], outs[b])
            start_out(c, b)

            @pl.when(c + 2 < _NCHUNK)
            def _():
                start_in(c + 2, b)
        return 0

    lax.fori_loop(0, _NCHUNK // 2, outer, 0)

    wait_out(_NCHUNK - 2, 0)
    wait_out(_NCHUNK - 1, 1)


_CBUF = pltpu.VMEM((_CROWS, _COLS), jnp.float32)

_sc_run = functools.partial(
    pl.kernel,
    out_type=jax.ShapeDtypeStruct((3, _ROWS, _COLS), jnp.float32),
    mesh=plsc.VectorSubcoreMesh(core_axis_name="c", subcore_axis_name="s",
                                num_cores=_NC, num_subcores=_NS),
    compiler_params=pltpu.CompilerParams(needs_layout_passes=False,
                                         use_tc_tiling_on_sc=False),
    scratch_types=[
        pltpu.VMEM((32 * 128,), jnp.float32),
        _CBUF, _CBUF, _CBUF, _CBUF, _CBUF, _CBUF,
        _CBUF, _CBUF, _CBUF, _CBUF, _CBUF, _CBUF,
        pltpu.SemaphoreType.DMA,
        pltpu.SemaphoreType.DMA,
        pltpu.SemaphoreType.DMA,
        pltpu.SemaphoreType.DMA,
    ],
)(_sc_body)


def kernel(x, z_boundaries, alpha_boundaries, r_boundaries_log, boundaries_abs_r):
    pack = _build_pack(z_boundaries.reshape(1, 4), alpha_boundaries,
                       r_boundaries_log, boundaries_abs_r)
    out = _sc_run(x.transpose(2, 0, 1), pack.reshape(-1))
    return out.transpose(1, 2, 0)


# speedup vs baseline: 1.1486x; 1.0356x over previous
import functools

import jax
import jax.numpy as jnp
from jax import lax
from jax.experimental import pallas as pl
from jax.experimental.pallas import tpu as pltpu
from jax.experimental.pallas import tpu_sc as plsc

_THRESH = 1e-10

_OFF_ALUT = 1 * 128
_OFF_RLUT = 6 * 128
_OFF_AB = 11 * 128
_OFF_RB = 16 * 128
_COL_ZB = 8

_NC, _NS, _L = 2, 16, 16
_NW = _NC * _NS
_ROWS, _COLS = 4096, 2048
_ROWS_W = _ROWS // _NW
_CROWS = 4
_NCHUNK = _ROWS_W // _CROWS
_TRIPS = _CROWS * _COLS
_GROUPS = _TRIPS // _L


def _lut_body(zb_ref, ab_ref, rb_ref, babs_ref, out_ref):
    zb = zb_ref[...]
    ab = ab_ref[...]
    rb = rb_ref[...]
    babs = babs_ref[...]

    ri = lax.broadcasted_iota(jnp.int32, (5, 6), 0)
    ci = lax.broadcasted_iota(jnp.int32, (5, 6), 1)
    min_r = 0.5 * jnp.sum(jnp.where((ri == 1) & (ci < 2), babs, 0.0))
    max_r = 0.5 * jnp.sum(jnp.where((ri == 4) & (ci >= 4), babs, 0.0))

    one1 = jnp.ones((1, 1), jnp.float32)
    zero1 = jnp.zeros((1, 1), jnp.float32)
    z5 = jnp.concatenate([zero1, 0.5 * (zb[:, 1:] + zb[:, :-1]), one1], axis=1)
    row0 = jnp.concatenate(
        [z5, jnp.zeros((1, 3), jnp.float32), zb, jnp.zeros((1, 116), jnp.float32)], axis=1)

    ones5 = jnp.ones((5, 1), jnp.float32)
    zeros5 = jnp.zeros((5, 1), jnp.float32)
    a_lut = jnp.concatenate(
        [zeros5, 0.5 * (ab[:, 1:] + ab[:, :-1]), ones5,
         jnp.zeros((5, 111), jnp.float32)], axis=1)
    r_mid = jnp.concatenate(
        [zeros5, 0.5 * (rb[:, 1:] + rb[:, :-1]), ones5], axis=1)
    r_lut = jnp.clip(jnp.log(r_mid * max_r + min_r + _THRESH), 0.0, 1.0)
    r_lut = jnp.concatenate([r_lut, jnp.zeros((5, 111), jnp.float32)], axis=1)
    ab_pad = jnp.concatenate([ab, jnp.zeros((5, 112), jnp.float32)], axis=1)
    rb_pad = jnp.concatenate([rb, jnp.zeros((5, 112), jnp.float32)], axis=1)

    out_ref[...] = jnp.concatenate(
        [row0, a_lut, r_lut, ab_pad, rb_pad, jnp.zeros((11, 128), jnp.float32)],
        axis=0)


def _build_pack(zb2, ab, rb, babs):
    return pl.pallas_call(
        _lut_body,
        out_shape=jax.ShapeDtypeStruct((32, 128), jnp.float32),
    )(zb2, ab, rb, babs)


def _search5(lut_v, off, v):
    pos = jnp.zeros((_L,), jnp.int32)
    for s in (8, 4, 2, 1):
        probe = plsc.load_gather(lut_v, [off + (pos + (s - 1))])
        pos = jnp.where(probe < v, pos + s, pos)
    probe = plsc.load_gather(lut_v, [off + pos])
    return jnp.where(probe < v, pos + 1, pos)


def _sc_body(x_hbm, pack_hbm, o_hbm, lut_v, zbin_v,
             iz0, ia0, ir0, iz1, ia1, ir1,
             oz0, oa0, or0, oz1, oa1, or1,
             sem_in0, sem_in1, sem_out0, sem_out1):
    cid = lax.axis_index("c")
    sid = lax.axis_index("s")
    wid = sid * _NC + cid
    base_row = wid * _ROWS_W

    pltpu.sync_copy(pack_hbm, lut_v)

    zeros16 = jnp.zeros((_L,), jnp.int32)
    zbs = [plsc.load_gather(lut_v, [zeros16 + (_COL_ZB + j)])
           for j in range(4)]

    ins = ((iz0, ia0, ir0), (iz1, ia1, ir1))
    outs = ((oz0, oa0, or0), (oz1, oa1, or1))
    sem_ins = (sem_in0, sem_in1)
    sem_outs = (sem_out0, sem_out1)

    def in_slice(c, ch):
        return x_hbm.at[ch, pl.ds(base_row + c * _CROWS, _CROWS)]

    def out_slice(c, ch):
        return o_hbm.at[ch, pl.ds(base_row + c * _CROWS, _CROWS)]

    def start_in(c, b):
        for ch in range(3):
            pltpu.async_copy(in_slice(c, ch), ins[b][ch], sem_ins[b])

    def wait_in(c, b):
        for ch in range(3):
            pltpu.make_async_copy(in_slice(c, ch), ins[b][ch], sem_ins[b]).wait()

    def start_out(c, b):
        for ch in range(3):
            pltpu.async_copy(outs[b][ch], out_slice(c, ch), sem_outs[b])

    def wait_out(c, b):
        for ch in range(3):
            pltpu.make_async_copy(outs[b][ch], out_slice(c, ch), sem_outs[b]).wait()

    start_in(0, 0)
    start_in(1, 1)

    def compute_chunk(bufs_in, bufs_out):
        bz, ba, br = bufs_in
        oz, oa, orr = bufs_out

        @plsc.parallel_loop(0, _GROUPS, step=1, unroll=4)
        def _pz(g):
            rr = lax.shift_right_logical(g, 7)
            cc = jnp.bitwise_and(g, 127) * _L
            z = bz[rr, pl.ds(cc, _L)]
            zbin = (z > zbs[0]).astype(jnp.int32)
            for j in range(1, 4):
                zbin = zbin + (z > zbs[j]).astype(jnp.int32)
            oz[rr, pl.ds(cc, _L)] = plsc.load_gather(lut_v, [zbin])
            zbin_v[rr, pl.ds(cc, _L)] = zbin * 128

        @plsc.parallel_loop(0, _GROUPS, step=1, unroll=4)
        def _pa(g):
            rr = lax.shift_right_logical(g, 7)
            cc = jnp.bitwise_and(g, 127) * _L
            a = ba[rr, pl.ds(cc, _L)]
            zoff = zbin_v[rr, pl.ds(cc, _L)]
            posa = _search5(lut_v, zoff + _OFF_AB, a)
            oa[rr, pl.ds(cc, _L)] = plsc.load_gather(
                lut_v, [(zoff + _OFF_ALUT) + posa])

        @plsc.parallel_loop(0, _GROUPS, step=1, unroll=4)
        def _pr(g):
            rr = lax.shift_right_logical(g, 7)
            cc = jnp.bitwise_and(g, 127) * _L
            r = br[rr, pl.ds(cc, _L)]
            zoff = zbin_v[rr, pl.ds(cc, _L)]
            posr = _search5(lut_v, zoff + _OFF_RB, r)
            orr[rr, pl.ds(cc, _L)] = plsc.load_gather(
                lut_v, [(zoff + _OFF_RLUT) + posr])

    def outer(i, _):
        for b in range(2):
            c = 2 * i + b
            wait_in(c, b)

            @pl.when(c >= 2)
            def _():
                wait_out(c - 2, b)

            compute_chunk(ins[b], outs[b])
            start_out(c, b)

            @pl.when(c + 2 < _NCHUNK)
            def _():
                start_in(c + 2, b)
        return 0

    lax.fori_loop(0, _NCHUNK // 2, outer, 0)

    wait_out(_NCHUNK - 2, 0)
    wait_out(_NCHUNK - 1, 1)


_CBUF = pltpu.VMEM((_CROWS, _COLS), jnp.float32)

_sc_run = functools.partial(
    pl.kernel,
    out_type=jax.ShapeDtypeStruct((3, _ROWS, _COLS), jnp.float32),
    mesh=plsc.VectorSubcoreMesh(core_axis_name="c", subcore_axis_name="s",
                                num_cores=_NC, num_subcores=_NS),
    compiler_params=pltpu.CompilerParams(needs_layout_passes=False,
                                         use_tc_tiling_on_sc=False),
    scratch_types=[
        pltpu.VMEM((32 * 128,), jnp.float32),
        pltpu.VMEM((_CROWS, _COLS), jnp.int32),
        _CBUF, _CBUF, _CBUF, _CBUF, _CBUF, _CBUF,
        _CBUF, _CBUF, _CBUF, _CBUF, _CBUF, _CBUF,
        pltpu.SemaphoreType.DMA,
        pltpu.SemaphoreType.DMA,
        pltpu.SemaphoreType.DMA,
        pltpu.SemaphoreType.DMA,
    ],
)(_sc_body)


def kernel(x, z_boundaries, alpha_boundaries, r_boundaries_log, boundaries_abs_r):
    pack = _build_pack(z_boundaries.reshape(1, 4), alpha_boundaries,
                       r_boundaries_log, boundaries_abs_r)
    out = _sc_run(x.transpose(2, 0, 1), pack.reshape(-1))
    return out.transpose(1, 2, 0)


# speedup vs baseline: 2.1472x; 1.8694x over previous
import functools

import jax
import jax.numpy as jnp
from jax import lax
from jax.experimental import pallas as pl
from jax.experimental.pallas import tpu as pltpu
from jax.experimental.pallas import tpu_sc as plsc

_THRESH = 1e-10

_OFF_ALUT = 1 * 128
_OFF_RLUT = 6 * 128
_OFF_AB = 11 * 128
_OFF_RB = 16 * 128
_COL_ZB = 8

_S_ZB = 8
_S_AB = 16
_S_RB = 101
_S_AL = 186
_S_RL = 271
_CPY = 356
_SCRAM = 1536

_NC, _NS, _L = 2, 16, 16
_NW = _NC * _NS
_ROWS, _COLS = 4096, 2048
_ROWS_W = _ROWS // _NW
_CROWS = 4
_NCHUNK = _ROWS_W // _CROWS
_TRIPS = _CROWS * _COLS
_GROUPS = _TRIPS // _L


def _lut_body(zb_ref, ab_ref, rb_ref, babs_ref, out_ref):
    zb = zb_ref[...]
    ab = ab_ref[...]
    rb = rb_ref[...]
    babs = babs_ref[...]

    ri = lax.broadcasted_iota(jnp.int32, (5, 6), 0)
    ci = lax.broadcasted_iota(jnp.int32, (5, 6), 1)
    min_r = 0.5 * jnp.sum(jnp.where((ri == 1) & (ci < 2), babs, 0.0))
    max_r = 0.5 * jnp.sum(jnp.where((ri == 4) & (ci >= 4), babs, 0.0))

    one1 = jnp.ones((1, 1), jnp.float32)
    zero1 = jnp.zeros((1, 1), jnp.float32)
    z5 = jnp.concatenate([zero1, 0.5 * (zb[:, 1:] + zb[:, :-1]), one1], axis=1)
    row0 = jnp.concatenate(
        [z5, jnp.zeros((1, 3), jnp.float32), zb, jnp.zeros((1, 116), jnp.float32)], axis=1)

    ones5 = jnp.ones((5, 1), jnp.float32)
    zeros5 = jnp.zeros((5, 1), jnp.float32)
    a_lut = jnp.concatenate(
        [zeros5, 0.5 * (ab[:, 1:] + ab[:, :-1]), ones5,
         jnp.zeros((5, 111), jnp.float32)], axis=1)
    r_mid = jnp.concatenate(
        [zeros5, 0.5 * (rb[:, 1:] + rb[:, :-1]), ones5], axis=1)
    r_lut = jnp.clip(jnp.log(r_mid * max_r + min_r + _THRESH), 0.0, 1.0)
    r_lut = jnp.concatenate([r_lut, jnp.zeros((5, 111), jnp.float32)], axis=1)
    ab_pad = jnp.concatenate([ab, jnp.zeros((5, 112), jnp.float32)], axis=1)
    rb_pad = jnp.concatenate([rb, jnp.zeros((5, 112), jnp.float32)], axis=1)

    out_ref[...] = jnp.concatenate(
        [row0, a_lut, r_lut, ab_pad, rb_pad, jnp.zeros((11, 128), jnp.float32)],
        axis=0)


def _build_pack(zb2, ab, rb, babs):
    return pl.pallas_call(
        _lut_body,
        out_shape=jax.ShapeDtypeStruct((32, 128), jnp.float32),
    )(zb2, ab, rb, babs)


def _search5(lut_v, off, v):
    pos = jnp.zeros((_L,), jnp.int32)
    for s in (8, 4, 2, 1):
        probe = plsc.load_gather(lut_v, [off + (pos + (s - 1))])
        pos = jnp.where(probe < v, pos + s, pos)
    probe = plsc.load_gather(lut_v, [off + pos])
    return jnp.where(probe < v, pos + 1, pos)


def _sc_body(x_hbm, pack_hbm, o_hbm, lut_v, scram_v,
             iz0, ia0, ir0, iz1, ia1, ir1,
             oz0, oa0, or0, oz1, oa1, or1,
             sem_in0, sem_in1, sem_out0, sem_out1):
    cid = lax.axis_index("c")
    sid = lax.axis_index("s")
    wid = sid * _NC + cid
    base_row = wid * _ROWS_W

    pltpu.sync_copy(pack_hbm, lut_v)

    i16 = lax.iota(jnp.int32, _L)

    def scram_row(dst, src):
        for h in (0, 16):
            vec = lut_v[pl.ds(src + h, _L)]
            plsc.store_scatter(scram_v, [i16 + (dst + h)], vec)

    for cpy in range(4):
        b = cpy * _CPY
        plsc.store_scatter(scram_v, [i16 + b], lut_v[pl.ds(0, _L)])
        for i in range(5):
            scram_row(b + _S_AB + 17 * i, _OFF_AB + 128 * i)
        for i in range(5):
            scram_row(b + _S_RB + 17 * i, _OFF_RB + 128 * i)
        for i in range(5):
            scram_row(b + _S_AL + 17 * i, _OFF_ALUT + 128 * i)
        for i in range(5):
            scram_row(b + _S_RL + 17 * i, _OFF_RLUT + 128 * i)

    laneb = jnp.bitwise_and(i16, 3) * _CPY
    abase = laneb + _S_AB
    rbase = laneb + _S_RB
    albase = laneb + _S_AL
    rlbase = laneb + _S_RL
    zbs = [plsc.load_gather(scram_v, [laneb + (_S_ZB + j)])
           for j in range(4)]

    ins = ((iz0, ia0, ir0), (iz1, ia1, ir1))
    outs = ((oz0, oa0, or0), (oz1, oa1, or1))
    sem_ins = (sem_in0, sem_in1)
    sem_outs = (sem_out0, sem_out1)

    def in_slice(c, ch):
        return x_hbm.at[ch, pl.ds(base_row + c * _CROWS, _CROWS)]

    def out_slice(c, ch):
        return o_hbm.at[ch, pl.ds(base_row + c * _CROWS, _CROWS)]

    def start_in(c, b):
        for ch in range(3):
            pltpu.async_copy(in_slice(c, ch), ins[b][ch], sem_ins[b])

    def wait_in(c, b):
        for ch in range(3):
            pltpu.make_async_copy(in_slice(c, ch), ins[b][ch], sem_ins[b]).wait()

    def start_out(c, b):
        for ch in range(3):
            pltpu.async_copy(outs[b][ch], out_slice(c, ch), sem_outs[b])

    def wait_out(c, b):
        for ch in range(3):
            pltpu.make_async_copy(outs[b][ch], out_slice(c, ch), sem_outs[b]).wait()

    start_in(0, 0)
    start_in(1, 1)

    def compute_chunk(bufs_in, bufs_out):
        bz, ba, br = bufs_in
        oz, oa, orr = bufs_out

        @plsc.parallel_loop(0, _GROUPS, step=1, unroll=2)
        def _grp(g):
            rr = lax.shift_right_logical(g, 7)
            cc = jnp.bitwise_and(g, 127) * _L
            z = bz[rr, pl.ds(cc, _L)]
            a = ba[rr, pl.ds(cc, _L)]
            r = br[rr, pl.ds(cc, _L)]

            zbin = (z > zbs[0]).astype(jnp.int32)
            for j in range(1, 4):
                zbin = zbin + (z > zbs[j]).astype(jnp.int32)
            z17 = zbin * 17

            new_z = plsc.load_gather(scram_v, [laneb + zbin])
            posa = _search5(scram_v, abase + z17, a)
            new_a = plsc.load_gather(scram_v, [(albase + z17) + posa])
            posr = _search5(scram_v, rbase + z17, r)
            new_r = plsc.load_gather(scram_v, [(rlbase + z17) + posr])

            oz[rr, pl.ds(cc, _L)] = new_z
            oa[rr, pl.ds(cc, _L)] = new_a
            orr[rr, pl.ds(cc, _L)] = new_r

    def outer(i, _):
        for b in range(2):
            c = 2 * i + b
            wait_in(c, b)

            @pl.when(c >= 2)
            def _():
                wait_out(c - 2, b)

            compute_chunk(ins[b], outs[b])
            start_out(c, b)

            @pl.when(c + 2 < _NCHUNK)
            def _():
                start_in(c + 2, b)
        return 0

    lax.fori_loop(0, _NCHUNK // 2, outer, 0)

    wait_out(_NCHUNK - 2, 0)
    wait_out(_NCHUNK - 1, 1)


_CBUF = pltpu.VMEM((_CROWS, _COLS), jnp.float32)

_sc_run = functools.partial(
    pl.kernel,
    out_type=jax.ShapeDtypeStruct((3, _ROWS, _COLS), jnp.float32),
    mesh=plsc.VectorSubcoreMesh(core_axis_name="c", subcore_axis_name="s",
                                num_cores=_NC, num_subcores=_NS),
    compiler_params=pltpu.CompilerParams(needs_layout_passes=False,
                                         use_tc_tiling_on_sc=False),
    scratch_types=[
        pltpu.VMEM((32 * 128,), jnp.float32),
        pltpu.VMEM((_SCRAM,), jnp.float32),
        _CBUF, _CBUF, _CBUF, _CBUF, _CBUF, _CBUF,
        _CBUF, _CBUF, _CBUF, _CBUF, _CBUF, _CBUF,
        pltpu.SemaphoreType.DMA,
        pltpu.SemaphoreType.DMA,
        pltpu.SemaphoreType.DMA,
        pltpu.SemaphoreType.DMA,
    ],
)(_sc_body)


def kernel(x, z_boundaries, alpha_boundaries, r_boundaries_log, boundaries_abs_r):
    pack = _build_pack(z_boundaries.reshape(1, 4), alpha_boundaries,
                       r_boundaries_log, boundaries_abs_r)
    out = _sc_run(x.transpose(2, 0, 1), pack.reshape(-1))
    return out.transpose(1, 2, 0)
